# baseline (speedup 1.0000x reference)
import jax
import jax.numpy as jnp
from jax import lax
from jax.experimental import pallas as pl
from jax.experimental.pallas import tpu as pltpu
from jax.experimental.pallas import tpu_sc as plsc

B = 16
C = 19
H = 256
W = 256
TH = 1024
TW = 1024
N_TOTAL = B * C * H * W

NW = 32
CELLS_PER_WORKER = (B * H * W) // NW
CELLS_PER_CHUNK = 2048
CHUNKS = CELLS_PER_WORKER // CELLS_PER_CHUNK
PRED_CHUNK = CELLS_PER_CHUNK * C


def _ds_body(t_ref, a_ref, s_ref, o_ref):
    x = t_ref[0].astype(jnp.float32)
    y = jax.lax.dot(
        a_ref[...], x, preferred_element_type=jnp.float32
    )
    z = jax.lax.dot(
        y, s_ref[...], preferred_element_type=jnp.float32
    )
    o_ref[0] = z.astype(jnp.int32)


def _tc_downsample(targets, rowsel, colsel):
    return pl.pallas_call(
        _ds_body,
        grid=(B, 2),
        in_specs=[
            pl.BlockSpec((1, TH // 2, TW), lambda b, r: (b, r, 0)),
            pl.BlockSpec((H // 2, TH // 2), lambda b, r: (0, 0)),
            pl.BlockSpec((TW, W), lambda b, r: (0, 0)),
        ],
        out_specs=pl.BlockSpec((1, H // 2, W), lambda b, r: (b, r, 0)),
        out_shape=jax.ShapeDtypeStruct((B, H, W), jnp.int32),
    )(targets, rowsel, colsel)


def _sc_body(preds_hbm, cds_hbm, out_hbm, prd_buf, c_buf, acc_buf, sem0, sem1):
    cid = lax.axis_index("c")
    sid = lax.axis_index("s")
    wid = cid * 16 + sid
    base_cell = wid * CELLS_PER_WORKER
    lane = lax.iota(jnp.int32, 16)
    sems = (sem0, sem1)

    def start(ci):
        slot = ci % 2
        bc = base_cell + ci * CELLS_PER_CHUNK
        h1 = pltpu.make_async_copy(
            cds_hbm.at[pl.ds(bc, CELLS_PER_CHUNK)],
            c_buf.at[pl.ds(slot * CELLS_PER_CHUNK, CELLS_PER_CHUNK)],
            sems[slot],
        )
        h2 = pltpu.make_async_copy(
            preds_hbm.at[pl.ds(bc * C, PRED_CHUNK)],
            prd_buf.at[pl.ds(slot * PRED_CHUNK, PRED_CHUNK)],
            sems[slot],
        )
        h1.start()
        h2.start()
        return (h1, h2)

    pending = {0: start(0)}
    acc = jnp.zeros((16,), jnp.float32)
    for ci in range(CHUNKS):
        slot = ci % 2
        if ci + 1 < CHUNKS:
            pending[ci + 1] = start(ci + 1)
        h1, h2 = pending.pop(ci)
        h1.wait()
        h2.wait()
        c_off = slot * CELLS_PER_CHUNK
        p_off = slot * PRED_CHUNK

        def j_body(j, acc_in, c_off=c_off, p_off=p_off):
            cell = j * 16 + lane
            cls = plsc.load_gather(c_buf, [c_off + cell])
            x = plsc.load_gather(prd_buf, [p_off + cell * C + cls])
            return acc_in + jnp.minimum(jnp.maximum(-x, -100.0), 100.0)

        acc = lax.fori_loop(0, CELLS_PER_CHUNK // 16, j_body, acc)

    acc_buf[...] = acc
    pltpu.sync_copy(acc_buf, out_hbm.at[wid])


def _sc_sparse_sum(preds_flat, cds_flat):
    mesh = plsc.VectorSubcoreMesh(core_axis_name="c", subcore_axis_name="s")
    f = pl.kernel(
        _sc_body,
        mesh=mesh,
        out_type=jax.ShapeDtypeStruct((NW, 16), jnp.float32),
        scratch_types=[
            pltpu.VMEM((2 * PRED_CHUNK,), jnp.float32),
            pltpu.VMEM((2 * CELLS_PER_CHUNK,), jnp.int32),
            pltpu.VMEM((16,), jnp.float32),
            pltpu.SemaphoreType.DMA,
            pltpu.SemaphoreType.DMA,
        ],
        compiler_params=pltpu.CompilerParams(needs_layout_passes=False),
    )
    return f(preds_flat, cds_flat)


def _tc_body(x_ref, o_ref):
    i = pl.program_id(0)
    x = x_ref[...]
    p = jax.nn.sigmoid(x)
    t = jnp.maximum(jnp.log1p(-p), -100.0)
    s = -jnp.sum(t)

    @pl.when(i == 0)
    def _():
        o_ref[0, 0] = s

    @pl.when(i != 0)
    def _():
        o_ref[0, 0] = o_ref[0, 0] + s


def _tc_dense_sum(preds2d):
    return pl.pallas_call(
        _tc_body,
        grid=(C,),
        in_specs=[pl.BlockSpec((B, H * W), lambda i: (i, 0))],
        out_specs=pl.BlockSpec(
            (1, 1), lambda i: (0, 0), memory_space=pltpu.SMEM
        ),
        out_shape=jax.ShapeDtypeStruct((1, 1), jnp.float32),
    )(preds2d)


def kernel(preds, targets):
    targets = targets.astype(jnp.int32)
    preds_flat = preds.reshape(-1)
    colsel = (jnp.arange(TW)[:, None] == 4 * jnp.arange(W)[None, :]).astype(
        jnp.float32
    )
    rowsel = (
        4 * jnp.arange(H // 2)[:, None] == jnp.arange(TH // 2)[None, :]
    ).astype(jnp.float32)
    cds = _tc_downsample(targets, rowsel, colsel)
    sc_parts = _sc_sparse_sum(preds_flat, cds.reshape(-1))
    tc_parts = _tc_dense_sum(preds.reshape(B * C, H * W))
    total = jnp.sum(tc_parts) + jnp.sum(sc_parts)
    return total / jnp.float32(N_TOTAL)


# speedup vs baseline: 1.3371x; 1.3371x over previous
import jax
import jax.numpy as jnp
from jax import lax
from jax.experimental import pallas as pl
from jax.experimental.pallas import tpu as pltpu
from jax.experimental.pallas import tpu_sc as plsc

B = 16
C = 19
H = 256
W = 256
TH = 1024
TW = 1024
N_TOTAL = B * C * H * W

NW = 32
CELLS_PER_WORKER = (B * H * W) // NW
CELLS_PER_CHUNK = 2048
CHUNKS = CELLS_PER_WORKER // CELLS_PER_CHUNK
PRED_CHUNK = CELLS_PER_CHUNK * C


def _ds_body(t_ref, a_ref, s_ref, o_ref):
    x = t_ref[0].astype(jnp.float32)
    y = jax.lax.dot(
        a_ref[...], x, preferred_element_type=jnp.float32
    )
    z = jax.lax.dot(
        y, s_ref[...], preferred_element_type=jnp.float32
    )
    o_ref[0] = z.astype(jnp.int32)


def _tc_downsample(targets, rowsel, colsel):
    return pl.pallas_call(
        _ds_body,
        grid=(B, 2),
        in_specs=[
            pl.BlockSpec((1, TH // 2, TW), lambda b, r: (b, r, 0)),
            pl.BlockSpec((H // 2, TH // 2), lambda b, r: (0, 0)),
            pl.BlockSpec((TW, W), lambda b, r: (0, 0)),
        ],
        out_specs=pl.BlockSpec((1, H // 2, W), lambda b, r: (b, r, 0)),
        out_shape=jax.ShapeDtypeStruct((B, H, W), jnp.int32),
    )(targets, rowsel, colsel)


def _sc_body(preds_hbm, cds_hbm, out_hbm, prd_buf, c_buf, acc_buf, sem0, sem1):
    cid = lax.axis_index("c")
    sid = lax.axis_index("s")
    wid = cid * 16 + sid
    base_cell = wid * CELLS_PER_WORKER
    lane = lax.iota(jnp.int32, 16)
    sems = (sem0, sem1)

    def start(ci):
        slot = ci % 2
        bc = base_cell + ci * CELLS_PER_CHUNK
        h1 = pltpu.make_async_copy(
            cds_hbm.at[pl.ds(bc, CELLS_PER_CHUNK)],
            c_buf.at[pl.ds(slot * CELLS_PER_CHUNK, CELLS_PER_CHUNK)],
            sems[slot],
        )
        h2 = pltpu.make_async_copy(
            preds_hbm.at[pl.ds(bc * C, PRED_CHUNK)],
            prd_buf.at[pl.ds(slot * PRED_CHUNK, PRED_CHUNK)],
            sems[slot],
        )
        h1.start()
        h2.start()
        return (h1, h2)

    pending = {0: start(0)}
    acc = jnp.zeros((16,), jnp.float32)
    for ci in range(CHUNKS):
        slot = ci % 2
        if ci + 1 < CHUNKS:
            pending[ci + 1] = start(ci + 1)
        h1, h2 = pending.pop(ci)
        h1.wait()
        h2.wait()
        c_off = slot * CELLS_PER_CHUNK
        p_off = slot * PRED_CHUNK

        def j_body(j, acc_in, c_off=c_off, p_off=p_off):
            cell = j * 16 + lane
            cls = plsc.load_gather(c_buf, [c_off + cell])
            x = plsc.load_gather(prd_buf, [p_off + cell * C + cls])
            return acc_in + jnp.minimum(jnp.maximum(-x, -100.0), 100.0)

        acc = lax.fori_loop(0, CELLS_PER_CHUNK // 16, j_body, acc)

    acc_buf[...] = acc
    pltpu.sync_copy(acc_buf, out_hbm.at[wid])


def _sc_sparse_sum(preds_flat, cds_flat):
    mesh = plsc.VectorSubcoreMesh(core_axis_name="c", subcore_axis_name="s")
    f = pl.kernel(
        _sc_body,
        mesh=mesh,
        out_type=jax.ShapeDtypeStruct((NW, 16), jnp.float32),
        scratch_types=[
            pltpu.VMEM((2 * PRED_CHUNK,), jnp.float32),
            pltpu.VMEM((2 * CELLS_PER_CHUNK,), jnp.int32),
            pltpu.VMEM((16,), jnp.float32),
            pltpu.SemaphoreType.DMA,
            pltpu.SemaphoreType.DMA,
        ],
        compiler_params=pltpu.CompilerParams(needs_layout_passes=False),
    )
    return f(preds_flat, cds_flat)


def _tc_body(x_ref, o_ref):
    i = pl.program_id(0)
    x = x_ref[...]
    p = jax.nn.sigmoid(x)
    t = jnp.maximum(jnp.log1p(-p), -100.0)
    s = -jnp.sum(t)

    @pl.when(i == 0)
    def _():
        o_ref[0, 0] = s

    @pl.when(i != 0)
    def _():
        o_ref[0, 0] = o_ref[0, 0] + s


def _tc_dense_sum(preds):
    return pl.pallas_call(
        _tc_body,
        grid=(B,),
        in_specs=[pl.BlockSpec((1, C, H, W), lambda i: (i, 0, 0, 0))],
        out_specs=pl.BlockSpec(
            (1, 1), lambda i: (0, 0), memory_space=pltpu.SMEM
        ),
        out_shape=jax.ShapeDtypeStruct((1, 1), jnp.float32),
    )(preds)


def kernel(preds, targets):
    targets = targets.astype(jnp.int32)
    preds_flat = preds.reshape(-1)
    colsel = (jnp.arange(TW)[:, None] == 4 * jnp.arange(W)[None, :]).astype(
        jnp.float32
    )
    rowsel = (
        4 * jnp.arange(H // 2)[:, None] == jnp.arange(TH // 2)[None, :]
    ).astype(jnp.float32)
    cds = _tc_downsample(targets, rowsel, colsel)
    sc_parts = _sc_sparse_sum(preds_flat, cds.reshape(-1))
    tc_parts = _tc_dense_sum(preds)
    total = jnp.sum(tc_parts) + jnp.sum(sc_parts)
    return total / jnp.float32(N_TOTAL)


# speedup vs baseline: 1.8130x; 1.3560x over previous
import jax
import jax.numpy as jnp
from jax import lax
from jax.experimental import pallas as pl
from jax.experimental.pallas import tpu as pltpu
from jax.experimental.pallas import tpu_sc as plsc

B = 16
C = 19
H = 256
W = 256
TH = 1024
TW = 1024
N_TOTAL = B * C * H * W

NW = 32


def _ds_body(t_ref, a_ref, s_ref, o_ref):
    x = t_ref[0].astype(jnp.float32)
    y = jax.lax.dot(
        a_ref[...], x, preferred_element_type=jnp.float32
    )
    z = jax.lax.dot(
        y, s_ref[...], preferred_element_type=jnp.float32
    )
    zi = z.astype(jnp.int32)
    o_ref[0, 0] = zi[:, :128]
    o_ref[0, 1] = zi[:, 128:]


def _tc_downsample(targets, rowsel, colsel):
    return pl.pallas_call(
        _ds_body,
        grid=(B, 2),
        in_specs=[
            pl.BlockSpec((1, TH // 2, TW), lambda b, r: (b, r, 0)),
            pl.BlockSpec((H // 2, TH // 2), lambda b, r: (0, 0)),
            pl.BlockSpec((TW, W), lambda b, r: (0, 0)),
        ],
        out_specs=pl.BlockSpec((1, 2, H // 2, 128), lambda b, r: (b, 0, r, 0)),
        out_shape=jax.ShapeDtypeStruct((B, 2, H, 128), jnp.int32),
    )(targets, rowsel, colsel)


CHUNK_CELLS = 2048
CHUNK_ROWS = (CHUNK_CELLS * C) // 256
CHUNKS_PER_IMG = (C * H * W) // (CHUNK_ROWS * 256)
CHUNKS_PER_WORKER = (B * CHUNKS_PER_IMG) // NW


def _sc_body(preds_hbm, cds_hbm, out_hbm,
             prd0, prd1, cb0, cb1, acc_buf, sem0, sem1):
    cid = lax.axis_index("c")
    sid = lax.axis_index("s")
    wid = cid * 16 + sid
    b = wid // 2
    half = wid % 2
    lane = lax.iota(jnp.int32, 16)
    prd = (prd0, prd1)
    cb = (cb0, cb1)
    sems = (sem0, sem1)

    def start(ci):
        slot = ci % 2
        k_img = half * CHUNKS_PER_WORKER + ci
        h1 = pltpu.make_async_copy(
            cds_hbm.at[b, 0, pl.ds(k_img * 8, 8)], cb[slot].at[0], sems[slot]
        )
        h2 = pltpu.make_async_copy(
            cds_hbm.at[b, 1, pl.ds(k_img * 8, 8)], cb[slot].at[1], sems[slot]
        )
        h3 = pltpu.make_async_copy(
            preds_hbm.at[b, pl.ds(k_img * CHUNK_ROWS, CHUNK_ROWS)],
            prd[slot],
            sems[slot],
        )
        h1.start()
        h2.start()
        h3.start()
        return (h1, h2, h3)

    pending = {0: start(0)}
    acc = jnp.zeros((16,), jnp.float32)
    for ci in range(CHUNKS_PER_WORKER):
        slot = ci % 2
        if ci + 1 < CHUNKS_PER_WORKER:
            pending[ci + 1] = start(ci + 1)
        for h in pending.pop(ci):
            h.wait()

        def j_body(j, acc_in, slot=slot):
            cell = j * 16 + lane
            hh = lax.shift_right_logical(cell, 8)
            w = lax.bitwise_and(cell, 255)
            wh = lax.shift_right_logical(w, 7)
            w1 = lax.bitwise_and(w, 127)
            cls = plsc.load_gather(cb[slot], [wh, hh, w1])
            ol = cell * C + cls
            lr = lax.shift_right_logical(ol, 8)
            lc = lax.bitwise_and(ol, 255)
            x = plsc.load_gather(prd[slot], [lr, lc])
            return acc_in + jnp.minimum(jnp.maximum(-x, -100.0), 100.0)

        acc = lax.fori_loop(0, CHUNK_CELLS // 16, j_body, acc)

    acc_buf[...] = acc
    pltpu.sync_copy(acc_buf, out_hbm.at[wid])


def _sc_sparse_sum(preds3, cds4):
    mesh = plsc.VectorSubcoreMesh(core_axis_name="c", subcore_axis_name="s")
    f = pl.kernel(
        _sc_body,
        mesh=mesh,
        out_type=jax.ShapeDtypeStruct((NW, 16), jnp.float32),
        scratch_types=[
            pltpu.VMEM((CHUNK_ROWS, 256), jnp.float32),
            pltpu.VMEM((CHUNK_ROWS, 256), jnp.float32),
            pltpu.VMEM((2, 8, 128), jnp.int32),
            pltpu.VMEM((2, 8, 128), jnp.int32),
            pltpu.VMEM((16,), jnp.float32),
            pltpu.SemaphoreType.DMA,
            pltpu.SemaphoreType.DMA,
        ],
        compiler_params=pltpu.CompilerParams(
            needs_layout_passes=False, use_tc_tiling_on_sc=True
        ),
    )
    return f(preds3, cds4)


def _tc_body(x_ref, o_ref):
    i = pl.program_id(0)
    x = x_ref[...]
    p = jax.nn.sigmoid(x)
    t = jnp.maximum(jnp.log1p(-p), -100.0)
    s = -jnp.sum(t)

    @pl.when(i == 0)
    def _():
        o_ref[0, 0] = s

    @pl.when(i != 0)
    def _():
        o_ref[0, 0] = o_ref[0, 0] + s


def _tc_dense_sum(preds):
    return pl.pallas_call(
        _tc_body,
        grid=(B,),
        in_specs=[pl.BlockSpec((1, C, H, W), lambda i: (i, 0, 0, 0))],
        out_specs=pl.BlockSpec(
            (1, 1), lambda i: (0, 0), memory_space=pltpu.SMEM
        ),
        out_shape=jax.ShapeDtypeStruct((1, 1), jnp.float32),
    )(preds)


def kernel(preds, targets):
    targets = targets.astype(jnp.int32)
    colsel = (jnp.arange(TW)[:, None] == 4 * jnp.arange(W)[None, :]).astype(
        jnp.float32
    )
    rowsel = (
        4 * jnp.arange(H // 2)[:, None] == jnp.arange(TH // 2)[None, :]
    ).astype(jnp.float32)
    cds = _tc_downsample(targets, rowsel, colsel)
    sc_parts = _sc_sparse_sum(preds.reshape(B, C * H, W), cds)
    tc_parts = _tc_dense_sum(preds)
    total = jnp.sum(tc_parts) + jnp.sum(sc_parts)
    return total / jnp.float32(N_TOTAL)


# speedup vs baseline: 2.6101x; 1.4396x over previous
import jax
import jax.numpy as jnp
from jax import lax
from jax.experimental import pallas as pl
from jax.experimental.pallas import tpu as pltpu
from jax.experimental.pallas import tpu_sc as plsc

B = 16
C = 19
H = 256
W = 256
TH = 1024
TW = 1024
N_TOTAL = B * C * H * W

NW = 32


def _ds_body(t_ref, a_ref, s_ref, o_ref):
    x = t_ref[0].astype(jnp.float32)
    y = jax.lax.dot(
        a_ref[...], x, preferred_element_type=jnp.float32
    )
    z = jax.lax.dot(
        y, s_ref[...], preferred_element_type=jnp.float32
    )
    zi = z.astype(jnp.int32)
    o_ref[0, 0] = zi[:, :128]
    o_ref[0, 1] = zi[:, 128:]


def _tc_downsample(targets, rowsel, colsel):
    return pl.pallas_call(
        _ds_body,
        grid=(B, 2),
        in_specs=[
            pl.BlockSpec((1, TH // 2, TW), lambda b, r: (b, r, 0)),
            pl.BlockSpec((H // 2, TH // 2), lambda b, r: (0, 0)),
            pl.BlockSpec((TW, W), lambda b, r: (0, 0)),
        ],
        out_specs=pl.BlockSpec((1, 2, H // 2, 128), lambda b, r: (b, 0, r, 0)),
        out_shape=jax.ShapeDtypeStruct((B, 2, H, 128), jnp.int32),
    )(targets, rowsel, colsel)


CHUNK_CELLS = 2048
CHUNK_ROWS = (CHUNK_CELLS * C) // 256
CHUNKS_PER_IMG = (C * H * W) // (CHUNK_ROWS * 256)
CHUNKS_PER_WORKER = (B * CHUNKS_PER_IMG) // NW


def _sc_body(preds_hbm, tgt_hbm, out_hbm,
             prd0, prd1, cb0, cb1, acc_buf, sem0, sem1):
    cid = lax.axis_index("c")
    sid = lax.axis_index("s")
    wid = cid * 16 + sid
    b = wid // 2
    half = wid % 2
    lane = lax.iota(jnp.int32, 16)
    prd = (prd0, prd1)
    cb = (cb0, cb1)
    sems = (sem0, sem1)

    def start(ci):
        slot = ci % 2
        k_img = half * CHUNKS_PER_WORKER + ci
        copies = [
            pltpu.make_async_copy(
                tgt_hbm.at[b, (k_img * 8 + r) * 4], cb[slot].at[r], sems[slot]
            )
            for r in range(8)
        ]
        copies.append(
            pltpu.make_async_copy(
                preds_hbm.at[b, pl.ds(k_img * CHUNK_ROWS, CHUNK_ROWS)],
                prd[slot],
                sems[slot],
            )
        )
        for cp in copies:
            cp.start()
        return copies

    pending = {0: start(0)}
    acc = jnp.zeros((16,), jnp.float32)
    for ci in range(CHUNKS_PER_WORKER):
        slot = ci % 2
        if ci + 1 < CHUNKS_PER_WORKER:
            pending[ci + 1] = start(ci + 1)
        for h in pending.pop(ci):
            h.wait()

        def j_body(j, acc_in, slot=slot):
            cell = j * 16 + lane
            hh = lax.shift_right_logical(cell, 8)
            w = lax.bitwise_and(cell, 255)
            cls = plsc.load_gather(cb[slot], [hh, w * 4])
            ol = cell * C + cls
            lr = lax.shift_right_logical(ol, 8)
            lc = lax.bitwise_and(ol, 255)
            x = plsc.load_gather(prd[slot], [lr, lc])
            return acc_in + jnp.minimum(jnp.maximum(-x, -100.0), 100.0)

        acc = lax.fori_loop(0, CHUNK_CELLS // 16, j_body, acc)

    acc_buf[...] = acc
    pltpu.sync_copy(acc_buf, out_hbm.at[wid])


def _sc_sparse_sum(preds3, targets):
    mesh = plsc.VectorSubcoreMesh(core_axis_name="c", subcore_axis_name="s")
    f = pl.kernel(
        _sc_body,
        mesh=mesh,
        out_type=jax.ShapeDtypeStruct((NW, 16), jnp.float32),
        scratch_types=[
            pltpu.VMEM((CHUNK_ROWS, 256), jnp.float32),
            pltpu.VMEM((CHUNK_ROWS, 256), jnp.float32),
            pltpu.VMEM((8, TW), jnp.int32),
            pltpu.VMEM((8, TW), jnp.int32),
            pltpu.VMEM((16,), jnp.float32),
            pltpu.SemaphoreType.DMA,
            pltpu.SemaphoreType.DMA,
        ],
        compiler_params=pltpu.CompilerParams(
            needs_layout_passes=False, use_tc_tiling_on_sc=True
        ),
    )
    return f(preds3, targets)


def _tc_body(x_ref, o_ref):
    i = pl.program_id(0)
    x = x_ref[...]
    p = jax.nn.sigmoid(x)
    t = jnp.maximum(jnp.log1p(-p), -100.0)
    s = -jnp.sum(t)

    @pl.when(i == 0)
    def _():
        o_ref[0, 0] = s

    @pl.when(i != 0)
    def _():
        o_ref[0, 0] = o_ref[0, 0] + s


def _tc_dense_sum(preds):
    return pl.pallas_call(
        _tc_body,
        grid=(B,),
        in_specs=[pl.BlockSpec((1, C, H, W), lambda i: (i, 0, 0, 0))],
        out_specs=pl.BlockSpec(
            (1, 1), lambda i: (0, 0), memory_space=pltpu.SMEM
        ),
        out_shape=jax.ShapeDtypeStruct((1, 1), jnp.float32),
    )(preds)


def kernel(preds, targets):
    targets = targets.astype(jnp.int32)
    sc_parts = _sc_sparse_sum(preds.reshape(B, C * H, W), targets)
    tc_parts = _tc_dense_sum(preds)
    total = jnp.sum(tc_parts) + jnp.sum(sc_parts)
    return total / jnp.float32(N_TOTAL)


# speedup vs baseline: 2.6955x; 1.0327x over previous
import jax
import jax.numpy as jnp
from jax import lax
from jax.experimental import pallas as pl
from jax.experimental.pallas import tpu as pltpu
from jax.experimental.pallas import tpu_sc as plsc

B = 16
C = 19
H = 256
W = 256
TH = 1024
TW = 1024
N_TOTAL = B * C * H * W

NW = 32


def _ds_body(t_ref, a_ref, s_ref, o_ref):
    x = t_ref[0].astype(jnp.float32)
    y = jax.lax.dot(
        a_ref[...], x, preferred_element_type=jnp.float32
    )
    z = jax.lax.dot(
        y, s_ref[...], preferred_element_type=jnp.float32
    )
    zi = z.astype(jnp.int32)
    o_ref[0, 0] = zi[:, :128]
    o_ref[0, 1] = zi[:, 128:]


def _tc_downsample(targets, rowsel, colsel):
    return pl.pallas_call(
        _ds_body,
        grid=(B, 2),
        in_specs=[
            pl.BlockSpec((1, TH // 2, TW), lambda b, r: (b, r, 0)),
            pl.BlockSpec((H // 2, TH // 2), lambda b, r: (0, 0)),
            pl.BlockSpec((TW, W), lambda b, r: (0, 0)),
        ],
        out_specs=pl.BlockSpec((1, 2, H // 2, 128), lambda b, r: (b, 0, r, 0)),
        out_shape=jax.ShapeDtypeStruct((B, 2, H, 128), jnp.int32),
    )(targets, rowsel, colsel)


CHUNK_CELLS = 2048
CHUNK_ROWS = (CHUNK_CELLS * C) // 256
CHUNKS_PER_IMG = (C * H * W) // (CHUNK_ROWS * 256)
CHUNKS_PER_WORKER = (B * CHUNKS_PER_IMG) // NW


def _sc_body(preds_hbm, tgt_hbm, out_hbm,
             prd0, prd1, cb0, cb1, acc_buf, sem0, sem1):
    cid = lax.axis_index("c")
    sid = lax.axis_index("s")
    wid = cid * 16 + sid
    b = wid // 2
    half = wid % 2
    lane = lax.iota(jnp.int32, 16)
    prd = (prd0, prd1)
    cb = (cb0, cb1)
    sems = (sem0, sem1)

    def start(ci):
        slot = ci % 2
        k_img = half * CHUNKS_PER_WORKER + ci
        copies = [
            pltpu.make_async_copy(
                tgt_hbm.at[b, (k_img * 8 + r) * 4], cb[slot].at[r], sems[slot]
            )
            for r in range(8)
        ]
        copies.append(
            pltpu.make_async_copy(
                preds_hbm.at[b, pl.ds(k_img * CHUNK_ROWS, CHUNK_ROWS)],
                prd[slot],
                sems[slot],
            )
        )
        for cp in copies:
            cp.start()
        return copies

    pending = {0: start(0)}
    acc = jnp.zeros((16,), jnp.float32)
    for ci in range(CHUNKS_PER_WORKER):
        slot = ci % 2
        if ci + 1 < CHUNKS_PER_WORKER:
            pending[ci + 1] = start(ci + 1)
        for h in pending.pop(ci):
            h.wait()

        def j_body(j, acc_in, slot=slot):
            cell = j * 16 + lane
            hh = lax.shift_right_logical(cell, 8)
            w = lax.bitwise_and(cell, 255)
            cls = plsc.load_gather(cb[slot], [hh, w * 4])
            ol = cell * C + cls
            lr = lax.shift_right_logical(ol, 8)
            lc = lax.bitwise_and(ol, 255)
            x = plsc.load_gather(prd[slot], [lr, lc])
            return acc_in + jnp.minimum(jnp.maximum(-x, -100.0), 100.0)

        acc = lax.fori_loop(0, CHUNK_CELLS // 16, j_body, acc)

    acc_buf[...] = acc
    pltpu.sync_copy(acc_buf, out_hbm.at[wid])


def _sc_sparse_sum(preds3, targets):
    mesh = plsc.VectorSubcoreMesh(core_axis_name="c", subcore_axis_name="s")
    f = pl.kernel(
        _sc_body,
        mesh=mesh,
        out_type=jax.ShapeDtypeStruct((NW, 16), jnp.float32),
        scratch_types=[
            pltpu.VMEM((CHUNK_ROWS, 256), jnp.float32),
            pltpu.VMEM((CHUNK_ROWS, 256), jnp.float32),
            pltpu.VMEM((8, TW), jnp.int32),
            pltpu.VMEM((8, TW), jnp.int32),
            pltpu.VMEM((16,), jnp.float32),
            pltpu.SemaphoreType.DMA,
            pltpu.SemaphoreType.DMA,
        ],
        compiler_params=pltpu.CompilerParams(
            needs_layout_passes=False, use_tc_tiling_on_sc=True
        ),
    )
    return f(preds3, targets)


def _tc_body(x_ref, o_ref):
    i = pl.program_id(0)
    x = x_ref[...]
    ax = jnp.abs(x)
    e = jnp.exp2(ax * (-1.4426950408889634))
    sp = jnp.maximum(x, 0.0) + 0.6931471805599453 * jnp.log2(1.0 + e)
    s = jnp.sum(jnp.minimum(sp, 100.0))

    @pl.when(i == 0)
    def _():
        o_ref[0, 0] = s

    @pl.when(i != 0)
    def _():
        o_ref[0, 0] = o_ref[0, 0] + s


def _tc_dense_sum(preds):
    return pl.pallas_call(
        _tc_body,
        grid=(B,),
        in_specs=[pl.BlockSpec((1, C, H, W), lambda i: (i, 0, 0, 0))],
        out_specs=pl.BlockSpec(
            (1, 1), lambda i: (0, 0), memory_space=pltpu.SMEM
        ),
        out_shape=jax.ShapeDtypeStruct((1, 1), jnp.float32),
    )(preds)


def kernel(preds, targets):
    targets = targets.astype(jnp.int32)
    sc_parts = _sc_sparse_sum(preds.reshape(B, C * H, W), targets)
    tc_parts = _tc_dense_sum(preds)
    total = jnp.sum(tc_parts) + jnp.sum(sc_parts)
    return total / jnp.float32(N_TOTAL)


# speedup vs baseline: 2.7014x; 1.0022x over previous
import jax
import jax.numpy as jnp
from jax import lax
from jax.experimental import pallas as pl
from jax.experimental.pallas import tpu as pltpu
from jax.experimental.pallas import tpu_sc as plsc

B = 16
C = 19
H = 256
W = 256
TH = 1024
TW = 1024
N_TOTAL = B * C * H * W

NW = 32


CHUNK_CELLS = 2048
CHUNK_ROWS = (CHUNK_CELLS * C) // 256
CHUNKS_PER_IMG = (C * H * W) // (CHUNK_ROWS * 256)
CHUNKS_PER_WORKER = (B * CHUNKS_PER_IMG) // NW


def _sc_body(preds_hbm, tgt_hbm, out_hbm,
             prd0, prd1, cb0, cb1, acc_buf, sem0, sem1):
    cid = lax.axis_index("c")
    sid = lax.axis_index("s")
    wid = cid * 16 + sid
    b = wid // 2
    half = wid % 2
    lane = lax.iota(jnp.int32, 16)
    prd = (prd0, prd1)
    cb = (cb0, cb1)
    sems = (sem0, sem1)

    def start(ci):
        slot = ci % 2
        k_img = half * CHUNKS_PER_WORKER + ci
        copies = [
            pltpu.make_async_copy(
                tgt_hbm.at[b, (k_img * 8 + r) * 4], cb[slot].at[r], sems[slot]
            )
            for r in range(8)
        ]
        copies.append(
            pltpu.make_async_copy(
                preds_hbm.at[b, pl.ds(k_img * CHUNK_ROWS, CHUNK_ROWS)],
                prd[slot],
                sems[slot],
            )
        )
        for cp in copies:
            cp.start()
        return copies

    pending = {0: start(0)}
    acc = jnp.zeros((16,), jnp.float32)
    for ci in range(CHUNKS_PER_WORKER):
        slot = ci % 2
        if ci + 1 < CHUNKS_PER_WORKER:
            pending[ci + 1] = start(ci + 1)
        for h in pending.pop(ci):
            h.wait()

        def j_body(j, acc_in, slot=slot):
            cell = j * 16 + lane
            hh = lax.shift_right_logical(cell, 8)
            w = lax.bitwise_and(cell, 255)
            cls = plsc.load_gather(cb[slot], [hh, w * 4])
            ol = cell * C + cls
            lr = lax.shift_right_logical(ol, 8)
            lc = lax.bitwise_and(ol, 255)
            x = plsc.load_gather(prd[slot], [lr, lc])
            return acc_in + jnp.minimum(jnp.maximum(-x, -100.0), 100.0)

        acc = lax.fori_loop(0, CHUNK_CELLS // 16, j_body, acc)

    acc_buf[...] = acc
    pltpu.sync_copy(acc_buf, out_hbm.at[wid])


def _sc_sparse_sum(preds3, targets):
    mesh = plsc.VectorSubcoreMesh(core_axis_name="c", subcore_axis_name="s")
    f = pl.kernel(
        _sc_body,
        mesh=mesh,
        out_type=jax.ShapeDtypeStruct((NW, 16), jnp.float32),
        scratch_types=[
            pltpu.VMEM((CHUNK_ROWS, 256), jnp.float32),
            pltpu.VMEM((CHUNK_ROWS, 256), jnp.float32),
            pltpu.VMEM((8, TW), jnp.int32),
            pltpu.VMEM((8, TW), jnp.int32),
            pltpu.VMEM((16,), jnp.float32),
            pltpu.SemaphoreType.DMA,
            pltpu.SemaphoreType.DMA,
        ],
        compiler_params=pltpu.CompilerParams(
            needs_layout_passes=False, use_tc_tiling_on_sc=True
        ),
    )
    return f(preds3, targets)


def _tc_body(x_ref, o_ref):
    i = pl.program_id(0)
    x = x_ref[...]
    ax = jnp.abs(x)
    e = jnp.exp2(ax * (-1.4426950408889634))
    sp = jnp.maximum(x, 0.0) + 0.6931471805599453 * jnp.log2(1.0 + e)
    s = jnp.sum(jnp.minimum(sp, 100.0))

    @pl.when(i == 0)
    def _():
        o_ref[0, 0] = s

    @pl.when(i != 0)
    def _():
        o_ref[0, 0] = o_ref[0, 0] + s


def _tc_dense_sum(preds):
    return pl.pallas_call(
        _tc_body,
        grid=(B,),
        in_specs=[pl.BlockSpec((1, C, H, W), lambda i: (i, 0, 0, 0))],
        out_specs=pl.BlockSpec(
            (1, 1), lambda i: (0, 0), memory_space=pltpu.SMEM
        ),
        out_shape=jax.ShapeDtypeStruct((1, 1), jnp.float32),
    )(preds)


def kernel(preds, targets):
    targets = targets.astype(jnp.int32)
    sc_parts = _sc_sparse_sum(preds.reshape(B, C * H, W), targets)
    tc_parts = _tc_dense_sum(preds)
    total = jnp.sum(tc_parts) + jnp.sum(sc_parts)
    return total / jnp.float32(N_TOTAL)
